# baseline (speedup 1.0000x reference)
import jax
import jax.numpy as jnp
from jax import lax
from jax.experimental import pallas as pl
from jax.experimental.pallas import tpu as pltpu
from jax.experimental.pallas import tpu_sc as plsc

_T = 8192
_SHIFT = 1000
_A = 1024
_R = _A - _SHIFT
_W = _T + _A - _SHIFT + 104
_ROWS = 16 * 128
_NW = 32
_RPW = _ROWS // _NW
_CHUNK = 8
_NCHUNK = _RPW // _CHUNK
_UNROLL = 8
_NVEC = _T // 16


def _sc_roll_body(x_hbm, out_hbm, buf):
    wid = lax.axis_index("s") * 2 + lax.axis_index("c")
    base = wid * _RPW

    def step(c, carry):
        r0 = base + c * _CHUNK
        pltpu.sync_copy(x_hbm.at[pl.ds(r0, _CHUNK), pl.ds(_T - _A, _A)],
                        buf.at[:, pl.ds(0, _A)])
        pltpu.sync_copy(x_hbm.at[pl.ds(r0, _CHUNK), pl.ds(0, _W - _A)],
                        buf.at[:, pl.ds(_A, _W - _A)])

        perm8 = ((lax.iota(jnp.int32, 16) + 8) % 16).reshape(16, 1)
        lanemask = lax.iota(jnp.int32, 16) < 8
        _gdn = lax.GatherDimensionNumbers(
            offset_dims=(), collapsed_slice_dims=(0,), start_index_map=(0,))

        def _rot8(v):
            return lax.gather(v, perm8, _gdn, (1,),
                              mode=lax.GatherScatterMode.PROMISE_IN_BOUNDS)

        for row in range(_CHUNK):
            def vec(k, vprev, row=row):
                base_col = pl.multiple_of(k * (_UNROLL * 16), _UNROLL * 16)
                for u in range(_UNROLL):
                    col = base_col + u * 16
                    vnext = buf[row, pl.ds(col + 32, 16)]
                    w = jnp.where(lanemask,
                                  _rot8(vprev),
                                  _rot8(vnext))
                    buf[row, pl.ds(col, 16)] = w
                    vprev = vnext
                return vprev

            v0 = buf[row, pl.ds(16, 16)]
            lax.fori_loop(0, _NVEC // _UNROLL, vec, v0)

        pltpu.sync_copy(buf.at[:, pl.ds(0, _T)],
                        out_hbm.at[pl.ds(r0, _CHUNK), :])
        return carry

    lax.fori_loop(0, _NCHUNK, step, 0)


@jax.jit
def kernel(x):
    rows = x.reshape(_ROWS, _T)
    out = pl.kernel(
        _sc_roll_body,
        out_type=jax.ShapeDtypeStruct((_ROWS, _T), jnp.float32),
        mesh=plsc.VectorSubcoreMesh(core_axis_name="c", subcore_axis_name="s"),
        scratch_types=[pltpu.VMEM((_CHUNK, _W), jnp.float32)],
    )(rows)
    return out.reshape(x.shape)


# speedup vs baseline: 2.1038x; 2.1038x over previous
import jax
import jax.numpy as jnp
from jax.experimental import pallas as pl
from jax.experimental.pallas import tpu as pltpu

_T = 8192
_SHIFT = 1000
_ROWS = 16 * 128
_BLOCK_ROWS = 256
_GRID = _ROWS // _BLOCK_ROWS


def _roll_body(x_ref, o_ref):
    o_ref[...] = pltpu.roll(x_ref[...], _SHIFT, axis=1)


@jax.jit
def kernel(x):
    rows = x.reshape(_ROWS, _T)
    out = pl.pallas_call(
        _roll_body,
        grid=(_GRID,),
        in_specs=[pl.BlockSpec((_BLOCK_ROWS, _T), lambda i: (i, 0))],
        out_specs=pl.BlockSpec((_BLOCK_ROWS, _T), lambda i: (i, 0)),
        out_shape=jax.ShapeDtypeStruct((_ROWS, _T), jnp.float32),
    )(rows)
    return out.reshape(x.shape)
